# baseline (speedup 1.0000x reference)
import functools

import jax
import jax.numpy as jnp
from jax.experimental import pallas as pl


def _router_body(x_ref, wr_ref, w_ref):
    xt = x_ref[...].astype(jnp.bfloat16)
    logits = jnp.dot(xt, wr_ref[...].astype(jnp.bfloat16),
                     preferred_element_type=jnp.float32)
    scores = jax.nn.sigmoid(logits)
    e = scores.shape[-1]
    iota = jax.lax.broadcasted_iota(jnp.int32, scores.shape, 1)
    i1 = jnp.argmax(scores, axis=1)[:, None]
    s1 = jnp.max(scores, axis=1)
    masked = jnp.where(iota == i1, -1.0, scores)
    i2 = jnp.argmax(masked, axis=1)[:, None]
    s2 = jnp.max(masked, axis=1)
    denom = s1 + s2 + 1e-20
    sel = jnp.logical_or(iota == i1, iota == i2)
    w_ref[...] = jnp.where(sel, scores, 0.0) / denom[:, None]


def _expert_body(nexp, x_ref, w_ref, w1_ref, w3_ref, w2_ref, o_ref):
    e = pl.program_id(1)
    xt = x_ref[...]
    a = jnp.dot(xt, w1_ref[0], preferred_element_type=jnp.float32)
    b = jnp.dot(xt, w3_ref[0], preferred_element_type=jnp.float32)
    h = (a * jax.nn.sigmoid(a) * b).astype(jnp.bfloat16)
    y = jnp.dot(h, w2_ref[0], preferred_element_type=jnp.float32)
    wt = w_ref[...]
    iota = jax.lax.broadcasted_iota(jnp.int32, wt.shape, 1)
    wcol = jnp.where(e < nexp,
                     jnp.sum(jnp.where(iota == e, wt, 0.0), axis=1),
                     1.0)
    y = y * wcol[:, None]

    @pl.when(e == 0)
    def _init():
        o_ref[...] = y

    @pl.when(e > 0)
    def _acc():
        o_ref[...] = o_ref[...] + y


def kernel(x, Wr, W1, W3, W2, Ws1, Ws3, Ws2):
    b, s, d = x.shape
    t = b * s
    nexp, _, dff = W1.shape
    flat = x.reshape(t, d)

    ns = Ws1.shape[1] // dff
    w1_all = jnp.concatenate(
        [W1, Ws1.reshape(d, ns, dff).transpose(1, 0, 2)], axis=0)
    w3_all = jnp.concatenate(
        [W3, Ws3.reshape(d, ns, dff).transpose(1, 0, 2)], axis=0)
    w2_all = jnp.concatenate([W2, Ws2.reshape(ns, dff, d)], axis=0)
    n_units = nexp + ns

    flat_b = flat.astype(jnp.bfloat16)
    w1_b = w1_all.astype(jnp.bfloat16)
    w3_b = w3_all.astype(jnp.bfloat16)
    w2_b = w2_all.astype(jnp.bfloat16)

    tr = min(512, t)
    w = pl.pallas_call(
        _router_body,
        grid=(t // tr,),
        in_specs=[
            pl.BlockSpec((tr, d), lambda i: (i, 0)),
            pl.BlockSpec((d, nexp), lambda i: (0, 0)),
        ],
        out_specs=pl.BlockSpec((tr, nexp), lambda i: (i, 0)),
        out_shape=jax.ShapeDtypeStruct((t, nexp), jnp.float32),
    )(flat, Wr)

    tt = min(512, t)
    out = pl.pallas_call(
        functools.partial(_expert_body, nexp),
        grid=(t // tt, n_units),
        in_specs=[
            pl.BlockSpec((tt, d), lambda i, e: (i, 0)),
            pl.BlockSpec((tt, nexp), lambda i, e: (i, 0)),
            pl.BlockSpec((1, d, dff), lambda i, e: (e, 0, 0)),
            pl.BlockSpec((1, d, dff), lambda i, e: (e, 0, 0)),
            pl.BlockSpec((1, dff, d), lambda i, e: (e, 0, 0)),
        ],
        out_specs=pl.BlockSpec((tt, d), lambda i, e: (i, 0)),
        out_shape=jax.ShapeDtypeStruct((t, d), jnp.float32),
    )(flat_b, w, w1_b, w3_b, w2_b)

    return out.reshape(b, s, d)


# speedup vs baseline: 1.2654x; 1.2654x over previous
import functools

import jax
import jax.numpy as jnp
from jax.experimental import pallas as pl
from jax.experimental.pallas import tpu as pltpu


def _router_body(x_ref, wr_ref, w_ref):
    xt = x_ref[...].astype(jnp.bfloat16)
    logits = jnp.dot(xt, wr_ref[...].astype(jnp.bfloat16),
                     preferred_element_type=jnp.float32)
    scores = jax.nn.sigmoid(logits)
    iota = jax.lax.broadcasted_iota(jnp.int32, scores.shape, 1)
    i1 = jnp.argmax(scores, axis=1)[:, None]
    s1 = jnp.max(scores, axis=1)
    masked = jnp.where(iota == i1, -1.0, scores)
    i2 = jnp.argmax(masked, axis=1)[:, None]
    s2 = jnp.max(masked, axis=1)
    denom = s1 + s2 + 1e-20
    sel = jnp.logical_or(iota == i1, iota == i2)
    w_ref[...] = jnp.where(sel, scores, 0.0) / denom[:, None]


def _gather_body(t, pos_ref, xin_ref, xout_ref):
    def body(j, carry):
        row = xin_ref[pl.ds(j, 1)]
        xout_ref[pl.ds(pos_ref[j], 1)] = row
        xout_ref[pl.ds(pos_ref[j + t], 1)] = row
        return carry

    jax.lax.fori_loop(0, t, body, 0)


def _grouped_body(ut_ref, ue_ref, fi_ref, va_ref,
                  xs_ref, e3_ref, w1_ref, w3_ref, w2_ref, o_ref):
    u = pl.program_id(0)
    xt = xs_ref[...].astype(jnp.bfloat16)
    a = jnp.dot(xt, w1_ref[0], preferred_element_type=jnp.float32)
    bm = jnp.dot(xt, w3_ref[0], preferred_element_type=jnp.float32)
    h = (a * jax.nn.sigmoid(a) * bm).astype(jnp.bfloat16)
    y = jnp.dot(h, w2_ref[0], preferred_element_type=jnp.float32)
    ue = ue_ref[u].astype(jnp.float32)
    ok = jnp.logical_and(e3_ref[0] == ue, va_ref[u] == 1)
    y = y * jnp.where(ok, 1.0, 0.0)

    @pl.when(fi_ref[u] == 1)
    def _init():
        o_ref[...] = y

    @pl.when(fi_ref[u] == 0)
    def _acc():
        o_ref[...] = o_ref[...] + y


def _combine_body(tc, pa_ref, pb_ref,
                  xb_ref, ws1_ref, ws3_ref, ws2_ref, wab_ref, os_ref,
                  o_ref, ga_ref, gb_ref):
    i = pl.program_id(0)
    base = i * tc

    def body(j, carry):
        ga_ref[pl.ds(j, 1)] = os_ref[pl.ds(pa_ref[base + j], 1)]
        gb_ref[pl.ds(j, 1)] = os_ref[pl.ds(pb_ref[base + j], 1)]
        return carry

    jax.lax.fori_loop(0, tc, body, 0)

    xt = xb_ref[...]
    a = jnp.dot(xt, ws1_ref[...], preferred_element_type=jnp.float32)
    bm = jnp.dot(xt, ws3_ref[...], preferred_element_type=jnp.float32)
    h = (a * jax.nn.sigmoid(a) * bm).astype(jnp.bfloat16)
    y = jnp.dot(h, ws2_ref[...], preferred_element_type=jnp.float32)
    wab = wab_ref[0]
    d = o_ref.shape[-1]
    ga = ga_ref[...].reshape(tc, d)
    gb = gb_ref[...].reshape(tc, d)
    o_ref[...] = y + ga * wab[:, 0:1] + gb * wab[:, 1:2]


def kernel(x, Wr, W1, W3, W2, Ws1, Ws3, Ws2):
    b, s, d = x.shape
    t = b * s
    nexp, _, dff = W1.shape
    na = 2 * t
    dsub = d // 128
    flat = x.reshape(t, d)
    flat_b = flat.astype(jnp.bfloat16)
    w1_b = W1.astype(jnp.bfloat16)
    w3_b = W3.astype(jnp.bfloat16)
    w2_b = W2.astype(jnp.bfloat16)
    ws1_b = Ws1.astype(jnp.bfloat16)
    ws3_b = Ws3.astype(jnp.bfloat16)
    ws2_b = Ws2.astype(jnp.bfloat16)

    tr = min(512, t)
    w = pl.pallas_call(
        _router_body,
        grid=(t // tr,),
        in_specs=[
            pl.BlockSpec((tr, d), lambda i: (i, 0)),
            pl.BlockSpec((d, nexp), lambda i: (0, 0)),
        ],
        out_specs=pl.BlockSpec((tr, nexp), lambda i: (i, 0)),
        out_shape=jax.ShapeDtypeStruct((t, nexp), jnp.float32),
    )(flat, Wr)

    i1 = jnp.argmax(w, axis=1)
    wa = jnp.max(w, axis=1)
    iota = jax.lax.broadcasted_iota(jnp.int32, w.shape, 1)
    masked = jnp.where(iota == i1[:, None], -1.0, w)
    i2 = jnp.argmax(masked, axis=1)
    wb = jnp.max(masked, axis=1)
    flat_e = jnp.concatenate([i1, i2]).astype(jnp.int32)

    oh = (flat_e[:, None] == jnp.arange(nexp)[None, :]).astype(jnp.float32)
    bl = 128
    nb = na // bl
    oh3 = oh.reshape(nb, bl, nexp)
    tril = jnp.tril(jnp.ones((bl, bl), jnp.float32), -1)
    within = jnp.einsum("ij,bjk->bik", tril, oh3)
    bsum = jnp.sum(oh3, axis=1)
    boff = jnp.concatenate(
        [jnp.zeros((1, nexp), jnp.float32), jnp.cumsum(bsum, axis=0)[:-1]])
    rank = (within + boff[:, None, :]).reshape(na, nexp)
    rank_j = jnp.sum(rank * oh, axis=1)
    counts = jnp.sum(bsum, axis=0)
    start = jnp.concatenate(
        [jnp.zeros((1,), jnp.float32), jnp.cumsum(counts)[:-1]])
    pos = (jnp.sum(oh * start[None, :], axis=1) + rank_j).astype(jnp.int32)

    starts_i = start.astype(jnp.int32)
    e_sorted = jnp.sum(
        (jnp.arange(na)[:, None] >= starts_i[None, :]).astype(jnp.int32),
        axis=1) - 1

    tg = 256
    nt = na // tg
    first_e = e_sorted[::tg]
    last_e = e_sorted[tg - 1::tg]
    c = last_e - first_e + 1
    cum_inc = jnp.cumsum(c)
    cum_exc = cum_inc - c
    n_units = nt + nexp - 1
    uu = jnp.arange(n_units)
    unit_tile = jnp.clip(
        jnp.sum((uu[:, None] >= cum_inc[None, :]).astype(jnp.int32), axis=1),
        0, nt - 1)
    valid = (uu < cum_inc[-1]).astype(jnp.int32)
    unit_expert = jnp.clip(
        first_e[unit_tile] + uu - cum_exc[unit_tile], 0, nexp - 1
    ).astype(jnp.int32)
    is_first = jnp.logical_and(uu == cum_exc[unit_tile], valid == 1
                               ).astype(jnp.int32)

    pa = pos[:t]
    pb = pos[t:]
    wab = jnp.stack([wa, wb], axis=-1)
    e3 = e_sorted.reshape(nt, tg, 1).astype(jnp.float32)

    x_sorted = pl.pallas_call(
        functools.partial(_gather_body, t),
        grid_spec=pltpu.PrefetchScalarGridSpec(
            num_scalar_prefetch=1,
            grid=(1,),
            in_specs=[pl.BlockSpec((t, dsub, 128), lambda i, pr: (0, 0, 0))],
            out_specs=pl.BlockSpec((na, dsub, 128), lambda i, pr: (0, 0, 0)),
        ),
        out_shape=jax.ShapeDtypeStruct((na, dsub, 128), jnp.float32),
    )(pos, flat.reshape(t, dsub, 128))

    out_sorted = pl.pallas_call(
        _grouped_body,
        grid_spec=pltpu.PrefetchScalarGridSpec(
            num_scalar_prefetch=4,
            grid=(n_units,),
            in_specs=[
                pl.BlockSpec((tg, d), lambda u, ut, ue, fi, va: (ut[u], 0)),
                pl.BlockSpec((1, tg, 1), lambda u, ut, ue, fi, va: (ut[u], 0, 0)),
                pl.BlockSpec((1, d, dff), lambda u, ut, ue, fi, va: (ue[u], 0, 0)),
                pl.BlockSpec((1, d, dff), lambda u, ut, ue, fi, va: (ue[u], 0, 0)),
                pl.BlockSpec((1, dff, d), lambda u, ut, ue, fi, va: (ue[u], 0, 0)),
            ],
            out_specs=pl.BlockSpec((tg, d), lambda u, ut, ue, fi, va: (ut[u], 0)),
        ),
        out_shape=jax.ShapeDtypeStruct((na, d), jnp.float32),
    )(unit_tile, unit_expert, is_first, valid,
      x_sorted.reshape(na, d), e3, w1_b, w3_b, w2_b)

    tc = min(512, t)
    sdff = Ws1.shape[1]
    out = pl.pallas_call(
        functools.partial(_combine_body, tc),
        grid_spec=pltpu.PrefetchScalarGridSpec(
            num_scalar_prefetch=2,
            grid=(t // tc,),
            in_specs=[
                pl.BlockSpec((tc, d), lambda i, pa, pb: (i, 0)),
                pl.BlockSpec((d, sdff), lambda i, pa, pb: (0, 0)),
                pl.BlockSpec((d, sdff), lambda i, pa, pb: (0, 0)),
                pl.BlockSpec((sdff, d), lambda i, pa, pb: (0, 0)),
                pl.BlockSpec((1, tc, 2), lambda i, pa, pb: (i, 0, 0)),
                pl.BlockSpec((na, dsub, 128), lambda i, pa, pb: (0, 0, 0)),
            ],
            out_specs=pl.BlockSpec((tc, d), lambda i, pa, pb: (i, 0)),
            scratch_shapes=[
                pltpu.VMEM((tc, dsub, 128), jnp.float32),
                pltpu.VMEM((tc, dsub, 128), jnp.float32),
            ],
        ),
        out_shape=jax.ShapeDtypeStruct((t, d), jnp.float32),
    )(pa, pb, flat_b, ws1_b, ws3_b, ws2_b,
      wab.reshape(t // tc, tc, 2), out_sorted.reshape(na, dsub, 128))

    return out.reshape(b, s, d)
